# baseline (speedup 1.0000x reference)
import functools

import jax
import jax.numpy as jnp
from jax import lax
from jax.experimental import pallas as pl
from jax.experimental.pallas import tpu as pltpu
from jax.experimental.pallas import tpu_sc as plsc

EMB = 64
HID = 64
OUT = 32
NC = 2
NS = 16
NW = NC * NS


def _sc_gather_pairs(emb2, idx_flat):
    b = idx_flat.shape[0]
    b_per_w = b // NW
    mesh = plsc.VectorSubcoreMesh(core_axis_name="c", subcore_axis_name="s")

    @functools.partial(
        pl.kernel,
        mesh=mesh,
        out_type=jax.ShapeDtypeStruct((b, 2 * EMB), jnp.float32),
        scratch_types=[
            pltpu.VMEM((512,), jnp.int32),
            pltpu.VMEM((512, 2 * EMB), jnp.float32),
            pltpu.SemaphoreType.DMA,
        ],
    )
    def gather_kernel(table_hbm, idx_hbm, out_hbm, idx_v, rows_v, sem):
        wid = lax.axis_index("s") * NC + lax.axis_index("c")
        base = wid * b_per_w

        @pl.loop(0, b_per_w, step=512)
        def _(off):
            pltpu.sync_copy(idx_hbm.at[pl.ds(base + off, 512)], idx_v)
            pltpu.async_copy(table_hbm.at[idx_v], rows_v, sem).wait()
            pltpu.sync_copy(rows_v, out_hbm.at[pl.ds(base + off, 512)])

    return gather_kernel(emb2, idx_flat)


def _mlp(g, p0, p1, w1aT, w1bT, b1, w2T, b2, w3T, b3):
    n = g.shape[0] // 2
    blk = 2048
    nb = n // blk

    def body(g0_ref, g1_ref, p0_ref, p1_ref, w1a_ref, w1b_ref, b1_ref,
             w2_ref, b2_ref, w3_ref, b3_ref, o_ref):
        g0 = g0_ref[...]
        g1 = g1_ref[...]
        x1 = jnp.where(p0_ref[...] != 0, g0[:, EMB:], g0[:, :EMB])
        x2 = jnp.where(p1_ref[...] != 0, g1[:, EMB:], g1[:, :EMB])
        a = jnp.dot(x1, w1a_ref[...], preferred_element_type=jnp.float32)
        a = a + jnp.dot(x2, w1b_ref[...], preferred_element_type=jnp.float32)
        a = jnp.maximum(a + b1_ref[...], 0.0)
        a = jnp.dot(a, w2_ref[...], preferred_element_type=jnp.float32) + b2_ref[...]
        a = jnp.maximum(a, 0.0)
        o_ref[...] = jnp.dot(a, w3_ref[...], preferred_element_type=jnp.float32) + b3_ref[...]

    full = lambda shape: pl.BlockSpec(shape, lambda i: (0, 0))
    return pl.pallas_call(
        body,
        grid=(nb,),
        in_specs=[
            pl.BlockSpec((blk, 2 * EMB), lambda i: (i, 0)),
            pl.BlockSpec((blk, 2 * EMB), lambda i: (i + nb, 0)),
            pl.BlockSpec((blk, 1), lambda i: (i, 0)),
            pl.BlockSpec((blk, 1), lambda i: (i, 0)),
            full((EMB, HID)),
            full((EMB, HID)),
            full((1, HID)),
            full((HID, HID)),
            full((1, HID)),
            full((HID, OUT)),
            full((1, OUT)),
        ],
        out_specs=pl.BlockSpec((blk, OUT), lambda i: (i, 0)),
        out_shape=jax.ShapeDtypeStruct((n, OUT), jnp.float32),
    )(g, g, p0, p1, w1aT, w1bT, b1, w2T, b2, w3T, b3)


def kernel(x, emb, w1, b1, w2, b2, w3, b3):
    xi = x.astype(jnp.int32)
    idx_flat = xi.T.reshape(-1)
    pair_idx = idx_flat // 2
    parity = idx_flat % 2
    n = xi.shape[0]
    emb2 = emb.reshape(emb.shape[0] // 2, 2 * EMB)
    g = _sc_gather_pairs(emb2, pair_idx)
    return _mlp(
        g,
        parity[:n].reshape(n, 1),
        parity[n:].reshape(n, 1),
        w1[:, :EMB].T,
        w1[:, EMB:].T,
        b1.reshape(1, HID),
        w2.T,
        b2.reshape(1, HID),
        w3.T,
        b3.reshape(1, OUT),
    )
